# baseline (speedup 1.0000x reference)
import functools

import jax
import jax.numpy as jnp
from jax import lax
from jax.experimental import pallas as pl
from jax.experimental.pallas import tpu as pltpu
from jax.experimental.pallas import tpu_sc as plsc

N = 10000
D = 128
DH = 64
E = 320000
NP = 10240
TILES = 16
CH = 160
EPT = CH * 128
EP = TILES * EPT
RPT = NP // TILES
HEADS = 16
HD = 64
HIDDEN = 256
NCLS = 40
R = 0.3
SLOPE = 0.2

_f32 = jnp.float32


def _sc_message_passing(xs, nidx, hidx):
    mesh = plsc.VectorSubcoreMesh(core_axis_name="c", subcore_axis_name="s")

    @functools.partial(
        pl.kernel,
        out_type=(
            jax.ShapeDtypeStruct((2, NP, DH), _f32),
            jax.ShapeDtypeStruct((NP,), _f32),
            jax.ShapeDtypeStruct((2 * NP, DH), _f32),
        ),
        mesh=mesh,
        scratch_types=[
            pltpu.VMEM((CH, 128), jnp.int32),
            pltpu.VMEM((CH, 128), jnp.int32),
            pltpu.VMEM((2, 128, DH), _f32),
            pltpu.VMEM((128,), _f32),
            pltpu.VMEM((128,), _f32),
            pltpu.VMEM((64, DH), _f32),
            pltpu.VMEM_SHARED((NP, DH), _f32),
            pltpu.VMEM_SHARED((NP,), _f32),
            pltpu.VMEM_SHARED((NP,), _f32),
            pltpu.SemaphoreType.DMA,
            pltpu.SemaphoreType.DMA,
            pltpu.SemaphoreType.DMA,
        ],
        compiler_params=pltpu.CompilerParams(use_tc_tiling_on_sc=False),
    )
    def k(xs_hbm, nidx_hbm, hidx_hbm, agg_out, ncnt_out, he_stage,
          idx_n, idx_h, rows, ones_v, cbuf, zbuf,
          acc, he_cnt, n_cnt, gsem, ssem, csem):
        c = lax.axis_index("c")
        s = lax.axis_index("s")
        off = c * NP
        base = s * RPT

        with jax.named_scope("sc_setup"):
            pltpu.sync_copy(nidx_hbm.at[s], idx_n)
            pltpu.sync_copy(hidx_hbm.at[s], idx_h)

            for q in range(8):
                ones_v[pl.ds(q * 16, 16)] = jnp.full((16,), 1.0, _f32)

            def zrow(r, carry):
                for q in range(4):
                    zbuf[r, pl.ds(q * 16, 16)] = jnp.zeros((16,), _f32)
                return carry
            lax.fori_loop(0, 64, zrow, 0)

            def adj1(j, carry):
                for q in range(8):
                    sl = pl.ds(q * 16, 16)
                    idx_n[j, sl] = idx_n[j, sl] + off
                return carry
            lax.fori_loop(0, CH, adj1, 0)

            def zcp(kk, carry):
                rb = base + kk * 64
                pltpu.sync_copy(zbuf, acc.at[pl.ds(rb, 64), :])
                pltpu.sync_copy(zbuf.at[0], he_cnt.at[pl.ds(rb, 64)])
                pltpu.sync_copy(zbuf.at[0], n_cnt.at[pl.ds(rb, 64)])
                return carry
            lax.fori_loop(0, RPT // 64, zcp, 0)

        plsc.subcore_barrier()

        def pipeline(src_hbm, gidx, sidx, tgt, cnt_tgt, cnt_on):
            pltpu.async_copy(src_hbm.at[gidx.at[0]], rows.at[0], gsem)

            def body(i, carry):
                for b in range(2):
                    j = 2 * i + b
                    nb = 1 - b
                    pltpu.make_async_copy(
                        src_hbm.at[gidx.at[j]], rows.at[b], gsem).wait()

                    @pl.when(j >= 1)
                    def _():
                        pltpu.make_async_copy(
                            rows.at[nb], tgt.at[sidx.at[j - 1]],
                            ssem).wait()

                    @pl.when(j + 1 < CH)
                    def _():
                        pltpu.async_copy(src_hbm.at[gidx.at[j + 1]],
                                         rows.at[nb], gsem)

                    pltpu.async_copy(rows.at[b], tgt.at[sidx.at[j]],
                                     ssem, add=True)
                    if cnt_on is None:
                        pltpu.async_copy(ones_v, cnt_tgt.at[sidx.at[j]],
                                         csem, add=True)
                    else:
                        @pl.when(cnt_on)
                        def _():
                            pltpu.async_copy(ones_v,
                                             cnt_tgt.at[sidx.at[j]],
                                             csem, add=True)
                return carry
            lax.fori_loop(0, CH // 2, body, 0)
            pltpu.make_async_copy(
                rows.at[1], tgt.at[sidx.at[CH - 1]], ssem).wait()

        def drain_cnt(cond):
            @pl.when(cond)
            def _():
                pltpu.make_async_copy(nidx_hbm.at[s], idx_n, csem).wait()

        with jax.named_scope("sc_pass1"):
            pipeline(xs_hbm, idx_n, idx_h, acc, he_cnt, None)
            drain_cnt(True)

        plsc.subcore_barrier()

        with jax.named_scope("sc_scale"):
            def adj2(j, carry):
                for q in range(8):
                    sl = pl.ds(q * 16, 16)
                    idx_n[j, sl] = idx_n[j, sl] - off
                    idx_h[j, sl] = idx_h[j, sl] + off
                return carry
            lax.fori_loop(0, CH, adj2, 0)

            def sc_chunk(kk, carry):
                rb = base + kk * 128
                pltpu.sync_copy(acc.at[pl.ds(rb, 128), :], rows.at[0])
                pltpu.sync_copy(he_cnt.at[pl.ds(rb, 128)], cbuf)
                for q in range(8):
                    sl = pl.ds(q * 16, 16)
                    cbuf[sl] = 1.0 / jnp.maximum(cbuf[sl], 1.0)

                def sgrp(g, carry2):
                    invvec = cbuf[pl.ds(g * 16, 16)]
                    for ri in range(16):
                        inv = invvec[ri]
                        r = g * 16 + ri
                        for q in range(4):
                            sl = pl.ds(q * 16, 16)
                            rows[0, r, sl] = rows[0, r, sl] * inv
                    return carry2
                lax.fori_loop(0, 8, sgrp, 0)
                pltpu.sync_copy(rows.at[0],
                                he_stage.at[pl.ds(off + rb, 128), :])
                return carry
            lax.fori_loop(0, RPT // 128, sc_chunk, 0)

        plsc.subcore_barrier()

        with jax.named_scope("sc_zero2"):
            def zcp2(kk, carry):
                pltpu.sync_copy(zbuf, acc.at[pl.ds(base + kk * 64, 64), :])
                return carry
            lax.fori_loop(0, RPT // 64, zcp2, 0)

        plsc.subcore_barrier()

        with jax.named_scope("sc_pass2"):
            pipeline(he_stage, idx_h, idx_n, acc, n_cnt, c == 1)
            drain_cnt(c == 1)

        plsc.subcore_barrier()

        with jax.named_scope("sc_writeback"):
            pltpu.sync_copy(acc.at[pl.ds(base, RPT), :],
                            agg_out.at[c, pl.ds(base, RPT), :])

            @pl.when(c == 1)
            def _():
                pltpu.sync_copy(n_cnt.at[pl.ds(base, RPT)],
                                ncnt_out.at[pl.ds(base, RPT)])

    return k(xs, nidx, hidx)


def _tc_body(x_ref, agg_ref, ncnt_ref, wv_ref, wo_ref, bo_ref, wc_ref,
             bc_ref, out_ref):
    xb = x_ref[...]
    a = jnp.concatenate([agg_ref[0], agg_ref[1]], axis=-1)
    inv = 1.0 / jnp.maximum(ncnt_ref[...], 1.0)
    z = R * xb + (1.0 - R) * (a * inv)
    h = jnp.dot(z, wv_ref[...], preferred_element_type=_f32)
    h = jnp.where(h > 0, h, SLOPE * h)
    hid = jnp.dot(h, wo_ref[...], preferred_element_type=_f32) + bo_ref[...]
    hid = jnp.where(hid > 0, hid, SLOPE * hid)
    out_ref[...] = jnp.dot(hid, wc_ref[...], preferred_element_type=_f32) \
        + bc_ref[...]


def _tc_mlp(x, agg, ncnt, wvt, wot, bo2, wct, bc2):
    bm = 400
    grid = (N // bm,)
    return pl.pallas_call(
        _tc_body,
        grid=grid,
        in_specs=[
            pl.BlockSpec((bm, D), lambda i: (i, 0)),
            pl.BlockSpec((2, bm, DH), lambda i: (0, i, 0)),
            pl.BlockSpec((bm, 1), lambda i: (i, 0)),
            pl.BlockSpec((D, HEADS * HD), lambda i: (0, 0)),
            pl.BlockSpec((HEADS * HD, HIDDEN), lambda i: (0, 0)),
            pl.BlockSpec((1, HIDDEN), lambda i: (0, 0)),
            pl.BlockSpec((HIDDEN, NCLS), lambda i: (0, 0)),
            pl.BlockSpec((1, NCLS), lambda i: (0, 0)),
        ],
        out_specs=pl.BlockSpec((bm, NCLS), lambda i: (i, 0)),
        out_shape=jax.ShapeDtypeStruct((N, NCLS), _f32),
    )(x, agg, ncnt, wvt, wot, bo2, wct, bc2)


def kernel(x, hyperedge_index, Wq, Wk, Wv, Wo, bo, Wc, bc):
    node_idx = hyperedge_index[0]
    he_idx = hyperedge_index[1]

    pad = EP - E
    padv = jnp.full((pad,), N, jnp.int32)
    nidx = jnp.concatenate([node_idx, padv]).reshape(TILES, CH, 128)
    hidx = jnp.concatenate([he_idx, padv]).reshape(TILES, CH, 128)

    xpad = jnp.zeros((NP, D), _f32).at[:N].set(x)
    xs = jnp.concatenate([xpad[:, :DH], xpad[:, DH:]], axis=0)

    agg, ncnt, _ = _sc_message_passing(xs, nidx, hidx)

    wvt = Wv.reshape(HEADS * HD, D).T
    wot = Wo.T
    wct = Wc.T
    out = _tc_mlp(x, agg, ncnt.reshape(NP, 1), wvt, wot,
                  bo.reshape(1, HIDDEN), wct, bc.reshape(1, NCLS))
    return out


# speedup vs baseline: 1.4224x; 1.4224x over previous
import functools

import jax
import jax.numpy as jnp
from jax import lax
from jax.experimental import pallas as pl
from jax.experimental.pallas import tpu as pltpu
from jax.experimental.pallas import tpu_sc as plsc

N = 10000
D = 128
DH = 64
E = 320000
NP = 10240
TILES = 16
CH = 158
EPT = CH * 128
EP = TILES * EPT
RPT = NP // TILES
HEADS = 16
HD = 64
HIDDEN = 256
NCLS = 40
R = 0.3
SLOPE = 0.2

_f32 = jnp.float32


def _sc_message_passing(xs, nidx, hidx):
    mesh = plsc.VectorSubcoreMesh(core_axis_name="c", subcore_axis_name="s")

    @functools.partial(
        pl.kernel,
        out_type=(
            jax.ShapeDtypeStruct((2, NP, DH), _f32),
            jax.ShapeDtypeStruct((NP,), _f32),
            jax.ShapeDtypeStruct((2 * NP, DH), _f32),
        ),
        mesh=mesh,
        scratch_types=[
            pltpu.VMEM((CH, 128), jnp.int32),
            pltpu.VMEM((CH, 128), jnp.int32),
            pltpu.VMEM((2, 128, DH), _f32),
            pltpu.VMEM((128,), _f32),
            pltpu.VMEM((128,), _f32),
            pltpu.VMEM((64, DH), _f32),
            pltpu.VMEM_SHARED((NP, DH), _f32),
            pltpu.VMEM_SHARED((NP,), _f32),
            pltpu.VMEM_SHARED((NP,), _f32),
            pltpu.SemaphoreType.DMA,
            pltpu.SemaphoreType.DMA,
            pltpu.SemaphoreType.DMA,
            pltpu.SemaphoreType.DMA,
        ],
        compiler_params=pltpu.CompilerParams(use_tc_tiling_on_sc=False),
    )
    def k(xs_hbm, nidx_hbm, hidx_hbm, agg_out, ncnt_out, he_stage,
          idx_n, idx_h, rows, ones_v, cbuf, zbuf,
          acc, he_cnt, n_cnt, gsem, ssem, csem, nsem):
        c = lax.axis_index("c")
        s = lax.axis_index("s")
        off = c * NP
        base = s * RPT

        with jax.named_scope("sc_setup"):
            pltpu.sync_copy(nidx_hbm.at[s], idx_n)
            pltpu.sync_copy(hidx_hbm.at[s], idx_h)

            for q in range(8):
                ones_v[pl.ds(q * 16, 16)] = jnp.full((16,), 1.0, _f32)

            def zrow(r, carry):
                for q in range(4):
                    zbuf[r, pl.ds(q * 16, 16)] = jnp.zeros((16,), _f32)
                return carry
            lax.fori_loop(0, 64, zrow, 0)

            def adj1(j, carry):
                for q in range(8):
                    sl = pl.ds(q * 16, 16)
                    idx_n[j, sl] = idx_n[j, sl] + off
                return carry
            lax.fori_loop(0, CH, adj1, 0)

            def zcp(kk, carry):
                rb = base + kk * 64
                pltpu.sync_copy(zbuf, acc.at[pl.ds(rb, 64), :])
                pltpu.sync_copy(zbuf.at[0], he_cnt.at[pl.ds(rb, 64)])
                pltpu.sync_copy(zbuf.at[0], n_cnt.at[pl.ds(rb, 64)])
                return carry
            lax.fori_loop(0, RPT // 64, zcp, 0)

        plsc.subcore_barrier()

        with jax.named_scope("sc_pass1"):
            pltpu.async_copy(xs_hbm.at[idx_n.at[0]], rows.at[0], gsem)

            def p1(i, carry):
                for b in range(2):
                    j = 2 * i + b
                    nb = 1 - b
                    pltpu.make_async_copy(
                        xs_hbm.at[idx_n.at[j]], rows.at[b], gsem).wait()

                    @pl.when(j >= 1)
                    def _():
                        pltpu.make_async_copy(
                            rows.at[nb], acc.at[idx_h.at[j - 1]],
                            ssem).wait()

                    @pl.when(j + 1 < CH)
                    def _():
                        pltpu.async_copy(xs_hbm.at[idx_n.at[j + 1]],
                                         rows.at[nb], gsem)

                    pltpu.async_copy(rows.at[b], acc.at[idx_h.at[j]],
                                     ssem, add=True)
                    pltpu.async_copy(ones_v, he_cnt.at[idx_h.at[j]],
                                     csem, add=True)

                    @pl.when(c == 0)
                    def _():
                        pltpu.async_copy(ones_v, n_cnt.at[idx_n.at[j]],
                                         nsem, add=True)
                return carry
            lax.fori_loop(0, CH // 2, p1, 0)

            pltpu.make_async_copy(
                rows.at[1], acc.at[idx_h.at[CH - 1]], ssem).wait()

            def dr1(j, carry):
                pltpu.make_async_copy(
                    ones_v, he_cnt.at[idx_h.at[j]], csem).wait()

                @pl.when(c == 0)
                def _():
                    pltpu.make_async_copy(
                        ones_v, n_cnt.at[idx_n.at[j]], nsem).wait()
                return carry
            lax.fori_loop(0, CH, dr1, 0)

        plsc.subcore_barrier()

        with jax.named_scope("sc_scale"):
            def adj2(j, carry):
                for q in range(8):
                    sl = pl.ds(q * 16, 16)
                    idx_n[j, sl] = idx_n[j, sl] - off
                    idx_h[j, sl] = idx_h[j, sl] + off
                return carry
            lax.fori_loop(0, CH, adj2, 0)

            def sc_chunk(kk, carry):
                rb = base + kk * 128
                pltpu.sync_copy(acc.at[pl.ds(rb, 128), :], rows.at[0])
                pltpu.sync_copy(he_cnt.at[pl.ds(rb, 128)], cbuf)
                for q in range(8):
                    sl = pl.ds(q * 16, 16)
                    cbuf[sl] = 1.0 / jnp.maximum(cbuf[sl], 1.0)

                def sgrp(g, carry2):
                    invvec = cbuf[pl.ds(g * 16, 16)]
                    for ri in range(16):
                        inv = invvec[ri]
                        r = g * 16 + ri
                        for q in range(4):
                            sl = pl.ds(q * 16, 16)
                            rows[0, r, sl] = rows[0, r, sl] * inv
                    return carry2
                lax.fori_loop(0, 8, sgrp, 0)
                pltpu.sync_copy(rows.at[0],
                                he_stage.at[pl.ds(off + rb, 128), :])
                return carry
            lax.fori_loop(0, RPT // 128, sc_chunk, 0)

        plsc.subcore_barrier()

        with jax.named_scope("sc_zero2"):
            def zcp2(kk, carry):
                pltpu.sync_copy(zbuf, acc.at[pl.ds(base + kk * 64, 64), :])
                return carry
            lax.fori_loop(0, RPT // 64, zcp2, 0)

        plsc.subcore_barrier()

        with jax.named_scope("sc_pass2"):
            pltpu.async_copy(he_stage.at[idx_h.at[0]], rows.at[0], gsem)

            def p2(i, carry):
                for b in range(2):
                    j = 2 * i + b
                    nb = 1 - b
                    pltpu.make_async_copy(
                        he_stage.at[idx_h.at[j]], rows.at[b], gsem).wait()

                    @pl.when(j >= 1)
                    def _():
                        pltpu.make_async_copy(
                            rows.at[nb], acc.at[idx_n.at[j - 1]],
                            ssem).wait()

                    @pl.when(j + 1 < CH)
                    def _():
                        pltpu.async_copy(he_stage.at[idx_h.at[j + 1]],
                                         rows.at[nb], gsem)

                    pltpu.async_copy(rows.at[b], acc.at[idx_n.at[j]],
                                     ssem, add=True)
                return carry
            lax.fori_loop(0, CH // 2, p2, 0)
            pltpu.make_async_copy(
                rows.at[1], acc.at[idx_n.at[CH - 1]], ssem).wait()

        plsc.subcore_barrier()

        with jax.named_scope("sc_writeback"):
            pltpu.sync_copy(acc.at[pl.ds(base, RPT), :],
                            agg_out.at[c, pl.ds(base, RPT), :])

            @pl.when(c == 0)
            def _():
                pltpu.sync_copy(n_cnt.at[pl.ds(base, RPT)],
                                ncnt_out.at[pl.ds(base, RPT)])

    return k(xs, nidx, hidx)


def _tc_body(x_ref, agg_ref, ncnt_ref, wv_ref, wo_ref, bo_ref, wc_ref,
             bc_ref, out_ref):
    xb = x_ref[...]
    a = jnp.concatenate([agg_ref[0], agg_ref[1]], axis=-1)
    inv = 1.0 / jnp.maximum(ncnt_ref[...], 1.0)
    z = R * xb + (1.0 - R) * (a * inv)
    h = jnp.dot(z, wv_ref[...], preferred_element_type=_f32)
    h = jnp.where(h > 0, h, SLOPE * h)
    hid = jnp.dot(h, wo_ref[...], preferred_element_type=_f32) + bo_ref[...]
    hid = jnp.where(hid > 0, hid, SLOPE * hid)
    out_ref[...] = jnp.dot(hid, wc_ref[...], preferred_element_type=_f32) \
        + bc_ref[...]


def _tc_mlp(x, agg, ncnt, wvt, wot, bo2, wct, bc2):
    bm = 400
    grid = (N // bm,)
    return pl.pallas_call(
        _tc_body,
        grid=grid,
        in_specs=[
            pl.BlockSpec((bm, D), lambda i: (i, 0)),
            pl.BlockSpec((2, bm, DH), lambda i: (0, i, 0)),
            pl.BlockSpec((bm, 1), lambda i: (i, 0)),
            pl.BlockSpec((D, HEADS * HD), lambda i: (0, 0)),
            pl.BlockSpec((HEADS * HD, HIDDEN), lambda i: (0, 0)),
            pl.BlockSpec((1, HIDDEN), lambda i: (0, 0)),
            pl.BlockSpec((HIDDEN, NCLS), lambda i: (0, 0)),
            pl.BlockSpec((1, NCLS), lambda i: (0, 0)),
        ],
        out_specs=pl.BlockSpec((bm, NCLS), lambda i: (i, 0)),
        out_shape=jax.ShapeDtypeStruct((N, NCLS), _f32),
    )(x, agg, ncnt, wvt, wot, bo2, wct, bc2)


def kernel(x, hyperedge_index, Wq, Wk, Wv, Wo, bo, Wc, bc):
    node_idx = hyperedge_index[0]
    he_idx = hyperedge_index[1]

    pad = EP - E
    padv = jnp.full((pad,), N, jnp.int32)
    nidx = jnp.concatenate([node_idx, padv]).reshape(TILES, CH, 128)
    hidx = jnp.concatenate([he_idx, padv]).reshape(TILES, CH, 128)

    xpad = jnp.zeros((NP, D), _f32).at[:N].set(x)
    xs = jnp.concatenate([xpad[:, :DH], xpad[:, DH:]], axis=0)

    agg, ncnt, _ = _sc_message_passing(xs, nidx, hidx)

    wvt = Wv.reshape(HEADS * HD, D).T
    wot = Wo.T
    wct = Wc.T
    out = _tc_mlp(x, agg, ncnt.reshape(NP, 1), wvt, wot,
                  bo.reshape(1, HIDDEN), wct, bc.reshape(1, NCLS))
    return out


# speedup vs baseline: 1.4230x; 1.0004x over previous
import functools

import jax
import jax.numpy as jnp
from jax import lax
from jax.experimental import pallas as pl
from jax.experimental.pallas import tpu as pltpu
from jax.experimental.pallas import tpu_sc as plsc

N = 10000
D = 128
DH = 64
E = 320000
NP = 10240
TILES = 16
CH = 158
EPT = CH * 128
EP = TILES * EPT
RPT = NP // TILES
HEADS = 16
HD = 64
HIDDEN = 256
NCLS = 40
R = 0.3
SLOPE = 0.2

_f32 = jnp.float32


def _sc_message_passing(xs, nidx, hidx):
    mesh = plsc.VectorSubcoreMesh(core_axis_name="c", subcore_axis_name="s")

    @functools.partial(
        pl.kernel,
        out_type=(
            jax.ShapeDtypeStruct((2, NP, DH), _f32),
            jax.ShapeDtypeStruct((NP,), _f32),
            jax.ShapeDtypeStruct((2 * NP, DH), _f32),
        ),
        mesh=mesh,
        scratch_types=[
            pltpu.VMEM((CH, 128), jnp.int32),
            pltpu.VMEM((CH, 128), jnp.int32),
            pltpu.VMEM((2, 128, DH), _f32),
            pltpu.VMEM((128,), _f32),
            pltpu.VMEM((128,), _f32),
            pltpu.VMEM((64, DH), _f32),
            pltpu.VMEM_SHARED((NP, DH), _f32),
            pltpu.VMEM_SHARED((NP,), _f32),
            pltpu.VMEM_SHARED((NP,), _f32),
            pltpu.SemaphoreType.DMA,
            pltpu.SemaphoreType.DMA,
            pltpu.SemaphoreType.DMA,
            pltpu.SemaphoreType.DMA,
        ],
        compiler_params=pltpu.CompilerParams(use_tc_tiling_on_sc=False),
    )
    def k(xs_hbm, nidx_hbm, hidx_hbm, agg_out, ncnt_out, he_stage,
          idx_n, idx_h, rows, ones_v, cbuf, zbuf,
          acc, he_cnt, n_cnt, gsem, ssem, csem, nsem):
        c = lax.axis_index("c")
        s = lax.axis_index("s")
        off = c * NP
        base = s * RPT

        with jax.named_scope("sc_setup"):
            pltpu.sync_copy(nidx_hbm.at[s], idx_n)
            pltpu.sync_copy(hidx_hbm.at[s], idx_h)

            for q in range(8):
                ones_v[pl.ds(q * 16, 16)] = jnp.full((16,), 1.0, _f32)

            def zrow(r, carry):
                for q in range(4):
                    zbuf[r, pl.ds(q * 16, 16)] = jnp.zeros((16,), _f32)
                return carry
            lax.fori_loop(0, 64, zrow, 0)

            def adj1(j, carry):
                for q in range(8):
                    sl = pl.ds(q * 16, 16)
                    idx_n[j, sl] = idx_n[j, sl] + off
                return carry
            lax.fori_loop(0, CH, adj1, 0)

            def zcp(kk, carry):
                rb = base + kk * 64
                pltpu.sync_copy(zbuf, acc.at[pl.ds(rb, 64), :])
                pltpu.sync_copy(zbuf.at[0], he_cnt.at[pl.ds(rb, 64)])
                pltpu.sync_copy(zbuf.at[0], n_cnt.at[pl.ds(rb, 64)])
                return carry
            lax.fori_loop(0, RPT // 64, zcp, 0)

        plsc.subcore_barrier()

        with jax.named_scope("sc_pass1"):
            pltpu.async_copy(xs_hbm.at[idx_n.at[0]], rows.at[0], gsem)

            def p1(i, carry):
                for b in range(2):
                    j = 2 * i + b
                    nb = 1 - b
                    pltpu.make_async_copy(
                        xs_hbm.at[idx_n.at[j]], rows.at[b], gsem).wait()

                    @pl.when(j >= 1)
                    def _():
                        pltpu.make_async_copy(
                            rows.at[nb], acc.at[idx_h.at[j - 1]],
                            ssem).wait()

                    @pl.when(j + 1 < CH)
                    def _():
                        pltpu.async_copy(xs_hbm.at[idx_n.at[j + 1]],
                                         rows.at[nb], gsem)

                    pltpu.async_copy(rows.at[b], acc.at[idx_h.at[j]],
                                     ssem, add=True)
                    pltpu.async_copy(ones_v, he_cnt.at[idx_h.at[j]],
                                     csem, add=True)

                    @pl.when(c == 0)
                    def _():
                        pltpu.async_copy(ones_v, n_cnt.at[idx_n.at[j]],
                                         nsem, add=True)
                return carry
            lax.fori_loop(0, CH // 2, p1, 0)

            pltpu.make_async_copy(
                rows.at[1], acc.at[idx_h.at[CH - 1]], ssem).wait()

            pltpu.make_async_copy(nidx_hbm.at[s], idx_n, csem).wait()

            @pl.when(c == 0)
            def _():
                pltpu.make_async_copy(nidx_hbm.at[s], idx_n, nsem).wait()

        plsc.subcore_barrier()

        with jax.named_scope("sc_scale"):
            def adj2(j, carry):
                for q in range(8):
                    sl = pl.ds(q * 16, 16)
                    idx_n[j, sl] = idx_n[j, sl] - off
                    idx_h[j, sl] = idx_h[j, sl] + off
                return carry
            lax.fori_loop(0, CH, adj2, 0)

            def sc_chunk(kk, carry):
                rb = base + kk * 128
                pltpu.sync_copy(acc.at[pl.ds(rb, 128), :], rows.at[0])
                pltpu.sync_copy(he_cnt.at[pl.ds(rb, 128)], cbuf)
                for q in range(8):
                    sl = pl.ds(q * 16, 16)
                    cbuf[sl] = 1.0 / jnp.maximum(cbuf[sl], 1.0)

                def sgrp(g, carry2):
                    invvec = cbuf[pl.ds(g * 16, 16)]
                    for ri in range(16):
                        inv = invvec[ri]
                        r = g * 16 + ri
                        for q in range(4):
                            sl = pl.ds(q * 16, 16)
                            rows[0, r, sl] = rows[0, r, sl] * inv
                    return carry2
                lax.fori_loop(0, 8, sgrp, 0)
                pltpu.sync_copy(rows.at[0],
                                he_stage.at[pl.ds(off + rb, 128), :])
                return carry
            lax.fori_loop(0, RPT // 128, sc_chunk, 0)

        plsc.subcore_barrier()

        with jax.named_scope("sc_zero2"):
            def zcp2(kk, carry):
                pltpu.sync_copy(zbuf, acc.at[pl.ds(base + kk * 64, 64), :])
                return carry
            lax.fori_loop(0, RPT // 64, zcp2, 0)

        plsc.subcore_barrier()

        with jax.named_scope("sc_pass2"):
            pltpu.async_copy(he_stage.at[idx_h.at[0]], rows.at[0], gsem)

            def p2(i, carry):
                for b in range(2):
                    j = 2 * i + b
                    nb = 1 - b
                    pltpu.make_async_copy(
                        he_stage.at[idx_h.at[j]], rows.at[b], gsem).wait()

                    @pl.when(j >= 1)
                    def _():
                        pltpu.make_async_copy(
                            rows.at[nb], acc.at[idx_n.at[j - 1]],
                            ssem).wait()

                    @pl.when(j + 1 < CH)
                    def _():
                        pltpu.async_copy(he_stage.at[idx_h.at[j + 1]],
                                         rows.at[nb], gsem)

                    pltpu.async_copy(rows.at[b], acc.at[idx_n.at[j]],
                                     ssem, add=True)
                return carry
            lax.fori_loop(0, CH // 2, p2, 0)
            pltpu.make_async_copy(
                rows.at[1], acc.at[idx_n.at[CH - 1]], ssem).wait()

        plsc.subcore_barrier()

        with jax.named_scope("sc_writeback"):
            pltpu.sync_copy(acc.at[pl.ds(base, RPT), :],
                            agg_out.at[c, pl.ds(base, RPT), :])

            @pl.when(c == 0)
            def _():
                pltpu.sync_copy(n_cnt.at[pl.ds(base, RPT)],
                                ncnt_out.at[pl.ds(base, RPT)])

    return k(xs, nidx, hidx)


def _tc_body(x_ref, agg_ref, ncnt_ref, wv_ref, wo_ref, bo_ref, wc_ref,
             bc_ref, out_ref):
    xb = x_ref[...]
    a = jnp.concatenate([agg_ref[0], agg_ref[1]], axis=-1)
    inv = 1.0 / jnp.maximum(ncnt_ref[...], 1.0)
    z = R * xb + (1.0 - R) * (a * inv)
    h = jnp.dot(z, wv_ref[...], preferred_element_type=_f32)
    h = jnp.where(h > 0, h, SLOPE * h)
    hid = jnp.dot(h, wo_ref[...], preferred_element_type=_f32) + bo_ref[...]
    hid = jnp.where(hid > 0, hid, SLOPE * hid)
    out_ref[...] = jnp.dot(hid, wc_ref[...], preferred_element_type=_f32) \
        + bc_ref[...]


def _tc_mlp(x, agg, ncnt, wvt, wot, bo2, wct, bc2):
    bm = 400
    grid = (N // bm,)
    return pl.pallas_call(
        _tc_body,
        grid=grid,
        in_specs=[
            pl.BlockSpec((bm, D), lambda i: (i, 0)),
            pl.BlockSpec((2, bm, DH), lambda i: (0, i, 0)),
            pl.BlockSpec((bm, 1), lambda i: (i, 0)),
            pl.BlockSpec((D, HEADS * HD), lambda i: (0, 0)),
            pl.BlockSpec((HEADS * HD, HIDDEN), lambda i: (0, 0)),
            pl.BlockSpec((1, HIDDEN), lambda i: (0, 0)),
            pl.BlockSpec((HIDDEN, NCLS), lambda i: (0, 0)),
            pl.BlockSpec((1, NCLS), lambda i: (0, 0)),
        ],
        out_specs=pl.BlockSpec((bm, NCLS), lambda i: (i, 0)),
        out_shape=jax.ShapeDtypeStruct((N, NCLS), _f32),
    )(x, agg, ncnt, wvt, wot, bo2, wct, bc2)


def kernel(x, hyperedge_index, Wq, Wk, Wv, Wo, bo, Wc, bc):
    node_idx = hyperedge_index[0]
    he_idx = hyperedge_index[1]

    pad = EP - E
    padv = jnp.full((pad,), N, jnp.int32)
    nidx = jnp.concatenate([node_idx, padv]).reshape(TILES, CH, 128)
    hidx = jnp.concatenate([he_idx, padv]).reshape(TILES, CH, 128)

    xpad = jnp.zeros((NP, D), _f32).at[:N].set(x)
    xs = jnp.concatenate([xpad[:, :DH], xpad[:, DH:]], axis=0)

    agg, ncnt, _ = _sc_message_passing(xs, nidx, hidx)

    wvt = Wv.reshape(HEADS * HD, D).T
    wot = Wo.T
    wct = Wc.T
    out = _tc_mlp(x, agg, ncnt.reshape(NP, 1), wvt, wot,
                  bo.reshape(1, HIDDEN), wct, bc.reshape(1, NCLS))
    return out
